# Initial kernel scaffold; baseline (speedup 1.0000x reference)
import jax
import jax.numpy as jnp
from jax.experimental import pallas as pl


def kernel(x, edge_index, edge_weight, W1, b1, W2, b2):
    raise NotImplementedError("write your pallas kernel here")


# speedup vs baseline: 32.0058x; 32.0058x over previous
import functools
import jax
import jax.numpy as jnp
from jax import lax
from jax.experimental import pallas as pl
from jax.experimental.pallas import tpu as pltpu
from jax.experimental.pallas import tpu_sc as plsc

N = 10000
NPAD = 10240
E = 320000
NC, NS = 2, 16
NW = NC * NS
PT = 10240
EPAD = PT * NW
CH = 2048
NCHUNK = PT // CH
SUB = 128
NSUB = CH // SUB
RPT = NPAD // NS
H = 16


def _sc_deg_body(col2_hbm, ew2_hbm, out_hbm, col_v, ew_v, zb_v, deg_sh, sem):
    c = lax.axis_index("c")
    s = lax.axis_index("s")

    def zero(i, _):
        zb_v[pl.ds(i * 16, 16)] = jnp.zeros((16,), jnp.float32)
        return 0

    lax.fori_loop(0, RPT // 16, zero, 0)
    pltpu.sync_copy(zb_v, deg_sh.at[pl.ds(pl.multiple_of(s * RPT, 8), RPT)])
    plsc.subcore_barrier()

    base_r = (c * NS + s) * (PT // SUB)

    def chunk(k, _):
        rb = pl.multiple_of(base_r + k * NSUB, 8)
        pltpu.sync_copy(col2_hbm.at[pl.ds(rb, NSUB)], col_v)
        pltpu.sync_copy(ew2_hbm.at[pl.ds(rb, NSUB)], ew_v)
        hs = [
            pltpu.async_copy(ew_v.at[j], deg_sh.at[col_v.at[j]], sem, add=True)
            for j in range(NSUB)
        ]
        for h in hs:
            h.wait()
        return 0

    lax.fori_loop(0, NCHUNK, chunk, 0)
    plsc.subcore_barrier()
    pltpu.sync_copy(deg_sh.at[pl.ds(pl.multiple_of(s * RPT, 8), RPT)], zb_v)
    pltpu.sync_copy(zb_v, out_hbm.at[c, pl.ds(pl.multiple_of(s * RPT, 8), RPT)])


def _sc_edge_body(y_hbm, row_hbm, col2_hbm, ew_hbm, out_hbm,
                  idx_v, col_v, ew_v, rows_v, zb_v, agg_sh, sem, gsem):
    c = lax.axis_index("c")
    s = lax.axis_index("s")

    def zero(i, _):
        zb_v[i, :] = jnp.zeros((16,), jnp.float32)
        return 0

    lax.fori_loop(0, RPT, zero, 0)
    pltpu.sync_copy(zb_v, agg_sh.at[pl.ds(pl.multiple_of(s * RPT, 8), RPT)])
    plsc.subcore_barrier()

    base = (c * NS + s) * PT

    def chunk(k, _):
        eb = pl.multiple_of(base + k * CH, 8)
        pltpu.sync_copy(row_hbm.at[pl.ds(eb, CH)], idx_v)
        pltpu.sync_copy(col2_hbm.at[pl.ds(pl.multiple_of(eb // SUB, 8), NSUB)], col_v)
        pltpu.sync_copy(ew_hbm.at[pl.ds(eb, CH)], ew_v)
        pltpu.async_copy(y_hbm.at[idx_v], rows_v, gsem).wait()

        def mul(g, _):
            e0 = g * 16
            w16 = ew_v[pl.ds(e0, 16)]
            for i in range(16):
                rows_v[e0 + i, :] = rows_v[e0 + i, :] * w16[i]
            return 0

        lax.fori_loop(0, CH // 16, mul, 0)
        hs = [
            pltpu.async_copy(rows_v.at[pl.ds(j * SUB, SUB)],
                             agg_sh.at[col_v.at[j]], sem, add=True)
            for j in range(NSUB)
        ]
        for h in hs:
            h.wait()
        return 0

    lax.fori_loop(0, NCHUNK, chunk, 0)
    plsc.subcore_barrier()
    pltpu.sync_copy(agg_sh.at[pl.ds(pl.multiple_of(s * RPT, 8), RPT)], zb_v)
    pltpu.sync_copy(zb_v, out_hbm.at[c, pl.ds(pl.multiple_of(s * RPT, 8), RPT)])


_sc_mesh = plsc.VectorSubcoreMesh(core_axis_name="c", subcore_axis_name="s")

_sc_deg = functools.partial(
    pl.kernel,
    out_type=jax.ShapeDtypeStruct((NC, NPAD), jnp.float32),
    mesh=_sc_mesh,
    compiler_params=pltpu.CompilerParams(use_tc_tiling_on_sc=False),
    scratch_types=[
        pltpu.VMEM((NSUB, SUB), jnp.int32),
        pltpu.VMEM((NSUB, SUB), jnp.float32),
        pltpu.VMEM((RPT,), jnp.float32),
        pltpu.VMEM_SHARED((NPAD,), jnp.float32),
        pltpu.SemaphoreType.DMA,
    ],
)(_sc_deg_body)

_sc_edge = functools.partial(
    pl.kernel,
    out_type=jax.ShapeDtypeStruct((NC, NPAD, H), jnp.float32),
    mesh=_sc_mesh,
    compiler_params=pltpu.CompilerParams(use_tc_tiling_on_sc=False),
    scratch_types=[
        pltpu.VMEM((CH,), jnp.int32),
        pltpu.VMEM((NSUB, SUB), jnp.int32),
        pltpu.VMEM((CH,), jnp.float32),
        pltpu.VMEM((CH, H), jnp.float32),
        pltpu.VMEM((RPT, H), jnp.float32),
        pltpu.VMEM_SHARED((NPAD, H), jnp.float32),
        pltpu.SemaphoreType.DMA,
        pltpu.SemaphoreType.DMA,
    ],
)(_sc_edge_body)


def _tc1_body(x_ref, w1_ref, d0_ref, d1_ref, y1_ref, dinv_ref):
    deg = d0_ref[...] + d1_ref[...] + 1.0
    dinv = jnp.where(deg > 0, lax.rsqrt(jnp.where(deg > 0, deg, 1.0)), 0.0)
    xw = jnp.dot(x_ref[...], w1_ref[...], preferred_element_type=jnp.float32)
    y1_ref[...] = dinv * xw
    dinv_ref[...] = dinv


def _tc2_body(a0_ref, a1_ref, y1_ref, dinv_ref, b1_ref, y2_ref):
    pre = dinv_ref[...] * (a0_ref[...] + a1_ref[...] + y1_ref[...]) + b1_ref[...]
    y2_ref[...] = dinv_ref[...] * jnp.maximum(pre, 0.0)


def _tc3_body(a0_ref, a1_ref, y2_ref, dinv_ref, w2_ref, b2_ref, out_ref):
    z = dinv_ref[...] * (a0_ref[...] + a1_ref[...] + y2_ref[...])
    out_ref[...] = (
        jnp.dot(z, w2_ref[...], preferred_element_type=jnp.float32) + b2_ref[...]
    )


def kernel(x, edge_index, edge_weight, W1, b1, W2, b2):
    row = edge_index[0]
    col = edge_index[1]
    pad = EPAD - E
    rowp = jnp.pad(row, (0, pad))
    colp = jnp.pad(col, (0, pad))
    ewp = jnp.pad(edge_weight, (0, pad))
    col2 = colp.reshape(EPAD // SUB, SUB)
    ew2 = ewp.reshape(EPAD // SUB, SUB)

    degp = _sc_deg(col2, ew2)
    d0 = degp[0, :N].reshape(N, 1)
    d1 = degp[1, :N].reshape(N, 1)

    y1, dinv = pl.pallas_call(
        _tc1_body,
        out_shape=[
            jax.ShapeDtypeStruct((N, H), jnp.float32),
            jax.ShapeDtypeStruct((N, 1), jnp.float32),
        ],
    )(x, W1, d0, d1)

    agg1 = _sc_edge(y1, rowp, col2, ewp)

    y2 = pl.pallas_call(
        _tc2_body,
        out_shape=jax.ShapeDtypeStruct((N, H), jnp.float32),
    )(agg1[0, :N], agg1[1, :N], y1, dinv, b1.reshape(1, H))

    agg2 = _sc_edge(y2, rowp, col2, ewp)

    out = pl.pallas_call(
        _tc3_body,
        out_shape=jax.ShapeDtypeStruct((N, 40), jnp.float32),
    )(agg2[0, :N], agg2[1, :N], y2, dinv, W2, b2.reshape(1, 40))
    return out
